# baseline (speedup 1.0000x reference)
import jax
import jax.numpy as jnp
from jax import lax
from jax.experimental import pallas as pl
from jax.experimental.pallas import tpu as pltpu
from jax.experimental.pallas import tpu_sc as plsc

VOCAB = 1000
EMBED = 32
BATCH = 4096
SEQ = 200

NC, NS, L = 2, 16, 16
NW = NC * NS
N = BATCH * SEQ
PER_W = N // NW
CH = 1024
NSTEPS = PER_W // CH


def _body(idx_hbm, tab_hbm, out_hbm, idx_v, rows_v, tab_v, sem_idx, sem_out):
    wid = lax.axis_index("s") * NC + lax.axis_index("c")
    base = wid * PER_W

    def idx_copy(g, buf):
        return pltpu.make_async_copy(
            idx_hbm.at[pl.ds(base + g * CH, CH)],
            idx_v.at[pl.ds(buf * CH, CH)],
            sem_idx,
        )

    def out_copy(g, buf):
        return pltpu.make_async_copy(
            rows_v.at[pl.ds(buf * CH * EMBED, CH * EMBED)],
            out_hbm.at[pl.ds((base + g * CH) * EMBED, CH * EMBED)],
            sem_out,
        )

    pltpu.sync_copy(tab_hbm, tab_v)
    idx_copy(0, 0).start()

    col_iota = lax.iota(jnp.int32, L) * EMBED

    def step(g, carry):
        buf = lax.rem(g, 2)
        idx_copy(g, buf).wait()

        @pl.when(g + 1 < NSTEPS)
        def _():
            idx_copy(g + 1, 1 - buf).start()

        @pl.when(g >= 2)
        def _():
            out_copy(g - 2, buf).wait()

        rbase = buf * CH

        def rows16(i, c2):
            iv = idx_v[pl.ds(rbase + i * L, L)]
            src = iv * EMBED
            dst = (rbase + i * L) * EMBED + col_iota
            for c in range(EMBED):
                vals = plsc.load_gather(tab_v, [src + c])
                plsc.store_scatter(rows_v, [dst + c], vals)
            return c2

        lax.fori_loop(0, CH // L, rows16, 0)

        out_copy(g, buf).start()
        return carry

    lax.fori_loop(0, NSTEPS, step, 0)
    out_copy(NSTEPS - 2, NSTEPS % 2).wait()
    out_copy(NSTEPS - 1, (NSTEPS - 1) % 2).wait()


@jax.jit
def _embed(idx_flat, tab_flat):
    mesh = plsc.VectorSubcoreMesh(core_axis_name="c", subcore_axis_name="s")
    run = pl.kernel(
        _body,
        out_type=jax.ShapeDtypeStruct((N * EMBED,), jnp.float32),
        mesh=mesh,
        scratch_types=[
            pltpu.VMEM((2 * CH,), jnp.int32),
            pltpu.VMEM((2 * CH * EMBED,), jnp.float32),
            pltpu.VMEM((VOCAB * EMBED,), jnp.float32),
            pltpu.SemaphoreType.DMA,
            pltpu.SemaphoreType.DMA,
        ],
        compiler_params=pltpu.CompilerParams(
            use_tc_tiling_on_sc=False, needs_layout_passes=False
        ),
    )
    return run(idx_flat, tab_flat)


def kernel(visit_segments, table):
    idx_flat = visit_segments.reshape(N).astype(jnp.int32)
    out = _embed(idx_flat, table.reshape(VOCAB * EMBED))
    return out.reshape(BATCH, SEQ, EMBED)


# speedup vs baseline: 2.5267x; 2.5267x over previous
import jax
import jax.numpy as jnp
from jax import lax
from jax.experimental import pallas as pl
from jax.experimental.pallas import tpu as pltpu
from jax.experimental.pallas import tpu_sc as plsc

VOCAB = 1000
EMBED = 32
BATCH = 4096
SEQ = 200

NC, NS = 2, 16
NW = NC * NS
N = BATCH * SEQ
PER_W = N // NW
CH = 1024
NSTEPS = PER_W // CH
SP = 560
HB = CH - SP


def _body(idx_hbm, tab_hbm, out_hbm, idx_v, rows_v, tab_sh,
          sem_idx, sem_gsp, sem_ghb, sem_out):
    wid = lax.axis_index("s") * NC + lax.axis_index("c")
    base = wid * PER_W

    def idx_copy(g):
        return pltpu.make_async_copy(
            idx_hbm.at[pl.ds(base + g * CH, CH)],
            idx_v.at[pl.ds(lax.rem(g, 4) * CH, CH)],
            sem_idx,
        )

    def gather_sp(g):
        o = lax.rem(g, 4) * CH
        r = lax.rem(g, 2) * CH
        return pltpu.make_async_copy(
            tab_sh.at[idx_v.at[pl.ds(o, SP)]],
            rows_v.at[pl.ds(r, SP)],
            sem_gsp,
        )

    def gather_hb(g):
        o = lax.rem(g, 4) * CH
        r = lax.rem(g, 2) * CH
        return pltpu.make_async_copy(
            tab_hbm.at[idx_v.at[pl.ds(o + SP, HB)]],
            rows_v.at[pl.ds(r + SP, HB)],
            sem_ghb,
        )

    def out_copy(g):
        r = lax.rem(g, 2) * CH
        return pltpu.make_async_copy(
            rows_v.at[pl.ds(r, CH)],
            out_hbm.at[pl.ds(base + g * CH, CH)],
            sem_out,
        )

    @pl.when(lax.axis_index("s") == 0)
    def _():
        pltpu.sync_copy(tab_hbm, tab_sh)

    idx_copy(0).start()
    plsc.subcore_barrier()

    def step(g, carry):
        idx_copy(g).wait()

        @pl.when(g >= 2)
        def _():
            out_copy(g - 2).wait()

        gather_sp(g).start()
        gather_hb(g).start()

        @pl.when(g + 1 < NSTEPS)
        def _():
            idx_copy(g + 1).start()

        @pl.when(g >= 1)
        def _():
            gather_sp(g - 1).wait()
            gather_hb(g - 1).wait()
            out_copy(g - 1).start()

        return carry

    lax.fori_loop(0, NSTEPS, step, 0)
    gather_sp(NSTEPS - 1).wait()
    gather_hb(NSTEPS - 1).wait()
    out_copy(NSTEPS - 1).start()
    out_copy(NSTEPS - 2).wait()
    out_copy(NSTEPS - 1).wait()


@jax.jit
def _embed(idx_flat, tab2d):
    mesh = plsc.VectorSubcoreMesh(core_axis_name="c", subcore_axis_name="s")
    run = pl.kernel(
        _body,
        out_type=jax.ShapeDtypeStruct((N, EMBED), jnp.float32),
        mesh=mesh,
        scratch_types=[
            pltpu.VMEM((4 * CH,), jnp.int32),
            pltpu.VMEM((2 * CH, EMBED), jnp.float32),
            pltpu.VMEM_SHARED((VOCAB, EMBED), jnp.float32),
            pltpu.SemaphoreType.DMA,
            pltpu.SemaphoreType.DMA,
            pltpu.SemaphoreType.DMA,
            pltpu.SemaphoreType.DMA,
        ],
        compiler_params=pltpu.CompilerParams(use_tc_tiling_on_sc=False),
    )
    return run(idx_flat, tab2d)


def kernel(visit_segments, table):
    idx_flat = visit_segments.reshape(N).astype(jnp.int32)
    out = _embed(idx_flat, table)
    return out.reshape(BATCH, SEQ, EMBED)
